# baseline (speedup 1.0000x reference)
import jax
import jax.numpy as jnp
from jax.experimental import pallas as pl
from jax.experimental.pallas import tpu as pltpu

_EPB = 2


def _moe_body(x_ref, w1a_ref, w1b_ref, w1c_ref, w1d_ref,
              w2a_ref, w2b_ref, w2c_ref, w2d_ref, tw_ref, ids_ref, out_ref):
    g = pl.program_id(0)
    x = x_ref[...]
    dn = (((1,), (1,)), ((), ()))
    kq = w2a_ref.shape[2]
    for i in range(_EPB):
        e = g * _EPB + i
        g1 = jax.lax.dot_general(x, w1a_ref[i, 0], dn,
                                 preferred_element_type=jnp.float32)
        g2 = jax.lax.dot_general(x, w1b_ref[i, 0], dn,
                                 preferred_element_type=jnp.float32)
        u1 = jax.lax.dot_general(x, w1c_ref[i, 0], dn,
                                 preferred_element_type=jnp.float32)
        u2 = jax.lax.dot_general(x, w1d_ref[i, 0], dn,
                                 preferred_element_type=jnp.float32)
        act = jnp.concatenate(
            [g1 * jax.nn.sigmoid(g1) * u1, g2 * jax.nn.sigmoid(g2) * u2],
            axis=1)
        sel = (ids_ref[...] == e).astype(jnp.float32)
        wpe = jnp.sum(tw_ref[...] * sel, axis=1, keepdims=True)
        for c, wr in enumerate((w2a_ref, w2b_ref, w2c_ref, w2d_ref)):
            yc = jax.lax.dot_general(act, wr[i, 0], dn,
                                     preferred_element_type=jnp.float32)
            if i == 0:
                @pl.when(g == 0)
                def _init(c=c, yc=yc):
                    out_ref[:, c * kq:(c + 1) * kq] = wpe * yc

                @pl.when(g > 0)
                def _acc(c=c, yc=yc):
                    out_ref[:, c * kq:(c + 1) * kq] += wpe * yc
            else:
                out_ref[:, c * kq:(c + 1) * kq] += wpe * yc


def kernel(hidden_states, w1, w2, topk_weights, topk_ids):
    m, k = hidden_states.shape
    e_total, two_n, _ = w1.shape
    n = w2.shape[2]
    topk = topk_ids.shape[1]
    nq = two_n // 4
    kq = k // 4
    w1r = w1.reshape(e_total, 4, nq, k)
    w2r = w2.reshape(e_total, 4, kq, n)
    w1spec = lambda q: pl.BlockSpec((_EPB, 1, nq, k),
                                    lambda g, q=q: (g, q, 0, 0))
    w2spec = lambda q: pl.BlockSpec((_EPB, 1, kq, n),
                                    lambda g, q=q: (g, q, 0, 0))
    return pl.pallas_call(
        _moe_body,
        grid=(e_total // _EPB,),
        in_specs=[
            pl.BlockSpec((m, k), lambda g: (0, 0)),
            w1spec(0), w1spec(1), w1spec(2), w1spec(3),
            w2spec(0), w2spec(1), w2spec(2), w2spec(3),
            pl.BlockSpec((m, topk), lambda g: (0, 0)),
            pl.BlockSpec((m, topk), lambda g: (0, 0)),
        ],
        out_specs=pl.BlockSpec((m, k), lambda g: (0, 0)),
        out_shape=jax.ShapeDtypeStruct((m, k), jnp.float32),
        compiler_params=pltpu.CompilerParams(
            dimension_semantics=("arbitrary",)),
    )(hidden_states, w1r, w1r, w1r, w1r, w2r, w2r, w2r, w2r,
      topk_weights, topk_ids)
